# baseline (speedup 1.0000x reference)
import functools

import numpy as np
import jax
import jax.numpy as jnp
from jax import lax
from jax.experimental import pallas as pl
from jax.experimental.pallas import tpu as pltpu
from jax.experimental.pallas import tpu_sc as plsc

B, U, W, V, D = 256, 20, 64, 30000, 128
NC, NS = 2, 16
NW = NC * NS
DPW = B // NW
ROWS = DPW * U
CG = D // 16


@functools.cache
def _make_sc_pool():
  mesh = plsc.VectorSubcoreMesh(core_axis_name="c", subcore_axis_name="s")

  @functools.partial(
      pl.kernel,
      mesh=mesh,
      out_type=jax.ShapeDtypeStruct((B * U, D), jnp.float32),
      scratch_types=[
          pltpu.VMEM((ROWS, 2 * W), jnp.int32),
          pltpu.VMEM((ROWS * W,), jnp.int32),
          pltpu.VMEM((ROWS + 16,), jnp.int32),
          pltpu.VMEM((ROWS,), jnp.int32),
          pltpu.VMEM((2, ROWS // 2), jnp.int32),
          pltpu.VMEM((8, W, D), jnp.float32),
          pltpu.VMEM((ROWS, D), jnp.float32),
          pltpu.SemaphoreType.DMA((8,)),
          pltpu.SemaphoreType.DMA,
      ],
  )
  def sc_pool(tok_hbm, eff_hbm, emb_hbm, out_hbm,
              idx_p, idx_v, eff_v, uidx, midx, buf, out_v, sem, sem_s):
    wid = lax.axis_index("s") * NC + lax.axis_index("c")
    base = wid * ROWS
    for t in range(ROWS // 16):
      lane = lax.iota(jnp.int32, 16)
      jv = lane * NW + (t * 16 * NW + wid)
      uidx[pl.ds(t * 16, 16)] = jv >> 1
      bv = (jv * 3277) >> 16
      mv = (jv - bv * U) * B + bv
      midx[t // 5, pl.ds((t % 5) * 16, 16)] = mv
    half = ROWS // 2
    for k in range(2):
      pltpu.make_async_copy(
          tok_hbm.at[uidx.at[pl.ds(k * half, half)]],
          idx_p.at[pl.ds(k * half, half)], sem_s).start()
    pltpu.sync_copy(eff_hbm.at[pl.ds(base, ROWS)], eff_v.at[pl.ds(0, ROWS)])
    for k in range(2):
      pltpu.make_async_copy(
          tok_hbm.at[uidx.at[pl.ds(k * half, half)]],
          idx_p.at[pl.ds(k * half, half)], sem_s).wait()
    half_off = (wid & 1) * W

    def compact(q, carry):
      for c in range(W // 16):
        idx_v[pl.ds(q * W + c * 16, 16)] = idx_p[q, pl.ds(half_off + c * 16, 16)]
      return carry

    lax.fori_loop(0, ROWS, compact, 0)

    def eff_at(j):
      return eff_v[pl.ds(j, 16)][0]

    def copy_for(j, nrows, s):
      return pltpu.make_async_copy(
          emb_hbm.at[idx_v.at[pl.ds(j * W, nrows)]],
          buf.at[s, pl.ds(0, nrows)], sem.at[s])

    zero16 = jnp.zeros((16,), jnp.float32)

    def issue(j, s):
      ne = (eff_at(j) + 7) >> 3
      for g in range(1, W // 8 + 1):
        @pl.when(ne == g)
        def _():
          copy_for(j, 8 * g, s).start()

    def wait_acc(j, s):
      eff = eff_at(j)
      ne = (eff + 7) >> 3
      for g in range(1, W // 8 + 1):
        @pl.when(ne == g)
        def _():
          copy_for(j, 8 * g, s).wait()
      nb = (eff + 15) >> 4
      def blk(i, acc):
        accs = list(acc)
        for rr in range(16):
          r = i * 16 + rr
          valid = r < eff
          for c in range(CG):
            accs[c] = accs[c] + jnp.where(valid, buf[s, r, pl.ds(c * 16, 16)],
                                          zero16)
        return tuple(accs)

      acc0 = tuple(jnp.zeros((16,), jnp.float32) for _ in range(CG))
      acc = lax.fori_loop(0, nb, blk, acc0)
      for c in range(CG):
        out_v[j, pl.ds(c * 16, 16)] = acc[c]

    for k in range(7):
      issue(k, k)

    def body(j, carry):
      @pl.when(j + 7 < ROWS)
      def _():
        issue(j + 7, lax.rem(j + 7, 8))

      wait_acc(j, lax.rem(j, 8))
      return carry

    lax.fori_loop(0, ROWS, body, 0)
    for k in range(2):
      pltpu.make_async_copy(
          out_v.at[pl.ds(k * half, half)],
          out_hbm.at[midx.at[k]], sem_s).start()
    for k in range(2):
      pltpu.make_async_copy(
          out_v.at[pl.ds(k * half, half)],
          out_hbm.at[midx.at[k]], sem_s).wait()

  return sc_pool


def _tc_body(x_ref, eff_ref, clen_ref, wu_ref, bu_ref, wd_ref, bd_ref, o_ref):
  eff = eff_ref[...]
  x = x_ref[...] / jnp.maximum(eff, 1).astype(jnp.float32)
  y = jnp.tanh(jnp.dot(x, wu_ref[...], preferred_element_type=jnp.float32)
               + bu_ref[...])
  y = y * (eff > 0).astype(jnp.float32)
  clen = clen_ref[...]
  inv = 1.0 / jnp.maximum(clen, 1).astype(jnp.float32)
  acc = jnp.zeros((B, D), jnp.float32)
  for u in range(U):
    wcol = jnp.where(u < clen, inv, 0.0)
    acc = acc + wcol * y[u * B:(u + 1) * B, :]
  c = jnp.tanh(jnp.dot(acc, wd_ref[...], preferred_element_type=jnp.float32)
               + bd_ref[...])
  o_ref[...] = c * (clen > 0).astype(jnp.float32)


_tc_finish = pl.pallas_call(
    _tc_body,
    out_shape=jax.ShapeDtypeStruct((B, D), jnp.float32),
)


def kernel(contexts, context_utterance_lengths, context_lengths,
           emb_table, W_u, b_u, W_d, b_d):
  contexts = contexts.astype(jnp.int32)
  cul = context_utterance_lengths.astype(jnp.int32)
  clen = context_lengths.astype(jnp.int32)
  upos = jnp.arange(U, dtype=jnp.int32)[None, :]
  eff = jnp.where(upos < clen[:, None], jnp.clip(cul, 0, W), 0)
  effp = eff.reshape(ROWS, NW).T.reshape(-1)
  sums_u = _make_sc_pool()(contexts.reshape(B * U // 2, 2 * W), effp,
                           emb_table.astype(jnp.float32))
  eff_t = eff.T.reshape(-1, 1)
  ctx = _tc_finish(sums_u, eff_t, clen[:, None],
                   W_u.astype(jnp.float32), b_u.astype(jnp.float32)[None, :],
                   W_d.astype(jnp.float32), b_d.astype(jnp.float32)[None, :])
  return ctx


# speedup vs baseline: 1.0113x; 1.0113x over previous
import functools

import numpy as np
import jax
import jax.numpy as jnp
from jax import lax
from jax.experimental import pallas as pl
from jax.experimental.pallas import tpu as pltpu
from jax.experimental.pallas import tpu_sc as plsc

B, U, W, V, D = 256, 20, 64, 30000, 128
NC, NS = 2, 16
NW = NC * NS
DPW = B // NW
ROWS = DPW * U
CG = D // 16


@functools.cache
def _make_sc_pool():
  mesh = plsc.VectorSubcoreMesh(core_axis_name="c", subcore_axis_name="s")

  @functools.partial(
      pl.kernel,
      mesh=mesh,
      out_type=jax.ShapeDtypeStruct((B * U, D), jnp.float32),
      scratch_types=[
          pltpu.VMEM((ROWS, 2 * W), jnp.int32),
          pltpu.VMEM((ROWS * W,), jnp.int32),
          pltpu.VMEM((ROWS + 16,), jnp.int32),
          pltpu.VMEM((ROWS,), jnp.int32),
          pltpu.VMEM((2, ROWS // 2), jnp.int32),
          pltpu.VMEM((8, W, D), jnp.float32),
          pltpu.VMEM((ROWS, D), jnp.float32),
          pltpu.SemaphoreType.DMA((8,)),
          pltpu.SemaphoreType.DMA,
      ],
  )
  def sc_pool(tok_hbm, eff_hbm, emb_hbm, out_hbm,
              idx_p, idx_v, eff_v, uidx, midx, buf, out_v, sem, sem_s):
    wid = lax.axis_index("s") * NC + lax.axis_index("c")
    base = wid * ROWS
    for t in range(ROWS // 16):
      lane = lax.iota(jnp.int32, 16)
      jv = lane * NW + (t * 16 * NW + wid)
      uidx[pl.ds(t * 16, 16)] = jv >> 1
      bv = (jv * 3277) >> 16
      mv = (jv - bv * U) * B + bv
      midx[t // 5, pl.ds((t % 5) * 16, 16)] = mv
    half = ROWS // 2
    for k in range(2):
      pltpu.make_async_copy(
          tok_hbm.at[uidx.at[pl.ds(k * half, half)]],
          idx_p.at[pl.ds(k * half, half)], sem_s).start()
    pltpu.sync_copy(eff_hbm.at[pl.ds(base, ROWS)], eff_v.at[pl.ds(0, ROWS)])
    for k in range(2):
      pltpu.make_async_copy(
          tok_hbm.at[uidx.at[pl.ds(k * half, half)]],
          idx_p.at[pl.ds(k * half, half)], sem_s).wait()
    half_off = (wid & 1) * W

    def compact(q, carry):
      for c in range(W // 16):
        idx_v[pl.ds(q * W + c * 16, 16)] = idx_p[q, pl.ds(half_off + c * 16, 16)]
      return carry

    lax.fori_loop(0, ROWS, compact, 0)

    def eff_at(j):
      return eff_v[pl.ds(j, 16)][0]

    def copy_for(j, nrows, s):
      return pltpu.make_async_copy(
          emb_hbm.at[idx_v.at[pl.ds(j * W, nrows)]],
          buf.at[s, pl.ds(0, nrows)], sem.at[s])

    zero16 = jnp.zeros((16,), jnp.float32)

    def issue(j, s):
      nb = (eff_at(j) + 15) >> 4
      for g in range(1, W // 16 + 1):
        @pl.when(nb == g)
        def _():
          copy_for(j, 16 * g, s).start()

    def wait_acc(j, s):
      eff = eff_at(j)
      nb = (eff + 15) >> 4
      for g in range(1, W // 16 + 1):
        @pl.when(nb == g)
        def _():
          copy_for(j, 16 * g, s).wait()
      def blk(i, acc):
        accs = list(acc)
        for rr in range(16):
          r = i * 16 + rr
          valid = r < eff
          for c in range(CG):
            accs[c] = accs[c] + jnp.where(valid, buf[s, r, pl.ds(c * 16, 16)],
                                          zero16)
        return tuple(accs)

      acc0 = tuple(jnp.zeros((16,), jnp.float32) for _ in range(CG))
      acc = lax.fori_loop(0, nb, blk, acc0)
      for c in range(CG):
        out_v[j, pl.ds(c * 16, 16)] = acc[c]

    for k in range(7):
      issue(k, k)

    def body(j, carry):
      @pl.when(j + 7 < ROWS)
      def _():
        issue(j + 7, lax.rem(j + 7, 8))

      wait_acc(j, lax.rem(j, 8))
      return carry

    lax.fori_loop(0, ROWS, body, 0)
    for k in range(2):
      pltpu.make_async_copy(
          out_v.at[pl.ds(k * half, half)],
          out_hbm.at[midx.at[k]], sem_s).start()
    for k in range(2):
      pltpu.make_async_copy(
          out_v.at[pl.ds(k * half, half)],
          out_hbm.at[midx.at[k]], sem_s).wait()

  return sc_pool


def _tc_body(x_ref, eff_ref, clen_ref, wu_ref, bu_ref, wd_ref, bd_ref, o_ref):
  eff = eff_ref[...]
  x = x_ref[...] / jnp.maximum(eff, 1).astype(jnp.float32)
  y = jnp.tanh(jnp.dot(x, wu_ref[...], preferred_element_type=jnp.float32)
               + bu_ref[...])
  y = y * (eff > 0).astype(jnp.float32)
  clen = clen_ref[...]
  inv = 1.0 / jnp.maximum(clen, 1).astype(jnp.float32)
  acc = jnp.zeros((B, D), jnp.float32)
  for u in range(U):
    wcol = jnp.where(u < clen, inv, 0.0)
    acc = acc + wcol * y[u * B:(u + 1) * B, :]
  c = jnp.tanh(jnp.dot(acc, wd_ref[...], preferred_element_type=jnp.float32)
               + bd_ref[...])
  o_ref[...] = c * (clen > 0).astype(jnp.float32)


_tc_finish = pl.pallas_call(
    _tc_body,
    out_shape=jax.ShapeDtypeStruct((B, D), jnp.float32),
)


def kernel(contexts, context_utterance_lengths, context_lengths,
           emb_table, W_u, b_u, W_d, b_d):
  contexts = contexts.astype(jnp.int32)
  cul = context_utterance_lengths.astype(jnp.int32)
  clen = context_lengths.astype(jnp.int32)
  upos = jnp.arange(U, dtype=jnp.int32)[None, :]
  eff = jnp.where(upos < clen[:, None], jnp.clip(cul, 0, W), 0)
  effp = eff.reshape(ROWS, NW).T.reshape(-1)
  sums_u = _make_sc_pool()(contexts.reshape(B * U // 2, 2 * W), effp,
                           emb_table.astype(jnp.float32))
  eff_t = eff.T.reshape(-1, 1)
  ctx = _tc_finish(sums_u, eff_t, clen[:, None],
                   W_u.astype(jnp.float32), b_u.astype(jnp.float32)[None, :],
                   W_d.astype(jnp.float32), b_d.astype(jnp.float32)[None, :])
  return ctx


# speedup vs baseline: 1.0172x; 1.0059x over previous
import functools

import numpy as np
import jax
import jax.numpy as jnp
from jax import lax
from jax.experimental import pallas as pl
from jax.experimental.pallas import tpu as pltpu
from jax.experimental.pallas import tpu_sc as plsc

B, U, W, V, D = 256, 20, 64, 30000, 128
NC, NS = 2, 16
NW = NC * NS
DPW = B // NW
ROWS = DPW * U
CG = D // 16


@functools.cache
def _make_sc_pool():
  mesh = plsc.VectorSubcoreMesh(core_axis_name="c", subcore_axis_name="s")

  @functools.partial(
      pl.kernel,
      mesh=mesh,
      out_type=jax.ShapeDtypeStruct((B * U, D), jnp.float32),
      scratch_types=[
          pltpu.VMEM((ROWS, 2 * W), jnp.int32),
          pltpu.VMEM((ROWS * W,), jnp.int32),
          pltpu.VMEM((ROWS + 16,), jnp.int32),
          pltpu.VMEM((ROWS,), jnp.int32),
          pltpu.VMEM((2, ROWS // 2), jnp.int32),
          pltpu.VMEM((8, W, D), jnp.float32),
          pltpu.VMEM((ROWS, D), jnp.float32),
          pltpu.SemaphoreType.DMA((8,)),
          pltpu.SemaphoreType.DMA,
      ],
  )
  def sc_pool(tok_hbm, eff_hbm, emb_hbm, out_hbm,
              idx_p, idx_v, eff_v, uidx, midx, buf, out_v, sem, sem_s):
    wid = lax.axis_index("s") * NC + lax.axis_index("c")
    base = wid * ROWS
    for t in range(ROWS // 16):
      lane = lax.iota(jnp.int32, 16)
      jv = lane * NW + (t * 16 * NW + wid)
      uidx[pl.ds(t * 16, 16)] = jv >> 1
      bv = (jv * 3277) >> 16
      mv = (jv - bv * U) * B + bv
      midx[t // 5, pl.ds((t % 5) * 16, 16)] = mv
    half = ROWS // 2
    for k in range(2):
      pltpu.make_async_copy(
          tok_hbm.at[uidx.at[pl.ds(k * half, half)]],
          idx_p.at[pl.ds(k * half, half)], sem_s).start()
    pltpu.sync_copy(eff_hbm.at[pl.ds(base, ROWS)], eff_v.at[pl.ds(0, ROWS)])
    for k in range(2):
      pltpu.make_async_copy(
          tok_hbm.at[uidx.at[pl.ds(k * half, half)]],
          idx_p.at[pl.ds(k * half, half)], sem_s).wait()
    half_off = (wid & 1) * W

    def compact(q, carry):
      for c in range(W // 16):
        idx_v[pl.ds(q * W + c * 16, 16)] = idx_p[q, pl.ds(half_off + c * 16, 16)]
      return carry

    lax.fori_loop(0, ROWS, compact, 0)

    def eff_at(j):
      return eff_v[pl.ds(j, 16)][0]

    def copy_for(j, nrows, s):
      return pltpu.make_async_copy(
          emb_hbm.at[idx_v.at[pl.ds(j * W, nrows)]],
          buf.at[s, pl.ds(0, nrows)], sem.at[s])

    def issue(j, s):
      nb = (eff_at(j) + 15) >> 4
      for g in range(1, W // 16 + 1):
        @pl.when(nb == g)
        def _():
          copy_for(j, 16 * g, s).start()

    def wait_acc(j, s):
      eff = eff_at(j)
      nb = (eff + 15) >> 4
      for g in range(1, W // 16 + 1):
        @pl.when(nb == g)
        def _():
          copy_for(j, 16 * g, s).wait()
      def blk(i, acc):
        accs = list(acc)
        for rr in range(16):
          r = i * 16 + rr
          wgt = (r < eff).astype(jnp.float32)
          for c in range(CG):
            accs[c] = accs[c] + wgt * buf[s, r, pl.ds(c * 16, 16)]
        return tuple(accs)

      acc0 = tuple(jnp.zeros((16,), jnp.float32) for _ in range(CG))
      acc = lax.fori_loop(0, nb, blk, acc0)
      for c in range(CG):
        out_v[j, pl.ds(c * 16, 16)] = acc[c]

    for k in range(7):
      issue(k, k)

    def body(j, carry):
      @pl.when(j + 7 < ROWS)
      def _():
        issue(j + 7, lax.rem(j + 7, 8))

      wait_acc(j, lax.rem(j, 8))
      return carry

    lax.fori_loop(0, ROWS, body, 0)
    for k in range(2):
      pltpu.make_async_copy(
          out_v.at[pl.ds(k * half, half)],
          out_hbm.at[midx.at[k]], sem_s).start()
    for k in range(2):
      pltpu.make_async_copy(
          out_v.at[pl.ds(k * half, half)],
          out_hbm.at[midx.at[k]], sem_s).wait()

  return sc_pool


def _tc_body(x_ref, eff_ref, clen_ref, wu_ref, bu_ref, wd_ref, bd_ref, o_ref):
  eff = eff_ref[...]
  x = x_ref[...] / jnp.maximum(eff, 1).astype(jnp.float32)
  y = jnp.tanh(jnp.dot(x, wu_ref[...], preferred_element_type=jnp.float32)
               + bu_ref[...])
  y = y * (eff > 0).astype(jnp.float32)
  clen = clen_ref[...]
  inv = 1.0 / jnp.maximum(clen, 1).astype(jnp.float32)
  acc = jnp.zeros((B, D), jnp.float32)
  for u in range(U):
    wcol = jnp.where(u < clen, inv, 0.0)
    acc = acc + wcol * y[u * B:(u + 1) * B, :]
  c = jnp.tanh(jnp.dot(acc, wd_ref[...], preferred_element_type=jnp.float32)
               + bd_ref[...])
  o_ref[...] = c * (clen > 0).astype(jnp.float32)


_tc_finish = pl.pallas_call(
    _tc_body,
    out_shape=jax.ShapeDtypeStruct((B, D), jnp.float32),
)


def kernel(contexts, context_utterance_lengths, context_lengths,
           emb_table, W_u, b_u, W_d, b_d):
  contexts = contexts.astype(jnp.int32)
  cul = context_utterance_lengths.astype(jnp.int32)
  clen = context_lengths.astype(jnp.int32)
  upos = jnp.arange(U, dtype=jnp.int32)[None, :]
  eff = jnp.where(upos < clen[:, None], jnp.clip(cul, 0, W), 0)
  effp = eff.reshape(ROWS, NW).T.reshape(-1)
  sums_u = _make_sc_pool()(contexts.reshape(B * U // 2, 2 * W), effp,
                           emb_table.astype(jnp.float32))
  eff_t = eff.T.reshape(-1, 1)
  ctx = _tc_finish(sums_u, eff_t, clen[:, None],
                   W_u.astype(jnp.float32), b_u.astype(jnp.float32)[None, :],
                   W_d.astype(jnp.float32), b_d.astype(jnp.float32)[None, :])
  return ctx
